# baseline (speedup 1.0000x reference)
import jax
import jax.numpy as jnp
from jax.experimental import pallas as pl
from jax.experimental.pallas import tpu as pltpu


def _fused_body(x_ref, w_ref, s_ref, b_ref, o_ref):
    nb = x_ref.shape[0] // 192
    x5 = x_ref[...].reshape(nb, 3, 8, 8, 64)
    x5 = x5.astype(jnp.bfloat16).astype(jnp.float32)
    s = jnp.sum(x5 * w_ref[...][None, :, None, :, :], axis=(1, 3))
    r = jnp.dot(s.reshape(nb * 8, 64), s_ref[...],
                preferred_element_type=jnp.float32)
    o_ref[...] = r + b_ref[...]


def kernel(conv_1_w, conv_1_b, conv_2_w, conv_2_b, conv_3_w, conv_3_b, x):
    N, C, H, W = x.shape
    Ho, Wo = H // 8, W // 8

    t = jnp.einsum("uckl,cvij->uvklij", conv_2_w, conv_1_w)
    wfull = jnp.einsum("upq,uvklij->vpkiqlj", conv_3_w[0], t).reshape(C, 8, 8)
    wfull = wfull.astype(jnp.bfloat16).astype(jnp.float32)
    wrow = jnp.tile(wfull, (1, 1, Wo))

    w3s = conv_3_w[0].sum(axis=(1, 2))
    bfull = (jnp.einsum("c,uckl,u->", conv_1_b, conv_2_w, w3s)
             + conv_2_b @ w3s + conv_3_b[0]).reshape(1, 1).astype(jnp.float32)

    S = (jnp.arange(W)[:, None] // 8 ==
         jnp.arange(Wo)[None, :]).astype(jnp.float32)

    xf = x.reshape(N * C * H, W)

    NB = 256
    while N % NB:
        NB //= 2
    out = pl.pallas_call(
        _fused_body,
        out_shape=jax.ShapeDtypeStruct((N * Ho, Wo), jnp.float32),
        grid_spec=pltpu.PrefetchScalarGridSpec(
            num_scalar_prefetch=0,
            grid=(N // NB,),
            in_specs=[
                pl.BlockSpec((NB * C * H, W), lambda i: (i, 0)),
                pl.BlockSpec((C, 8, W), lambda i: (0, 0, 0)),
                pl.BlockSpec((W, Wo), lambda i: (0, 0)),
                pl.BlockSpec((1, 1), lambda i: (0, 0)),
            ],
            out_specs=pl.BlockSpec((NB * Ho, Wo), lambda i: (i, 0)),
        ),
        compiler_params=pltpu.CompilerParams(
            dimension_semantics=("parallel",),
            vmem_limit_bytes=64 * 1024 * 1024,
        ),
    )(xf, wrow, S, bfull)

    return out.reshape(N, 1, Ho, Wo).astype(x.dtype)


# speedup vs baseline: 1.0678x; 1.0678x over previous
import jax
import jax.numpy as jnp
from jax.experimental import pallas as pl
from jax.experimental.pallas import tpu as pltpu


def _fused_body(x_ref, w_ref, s_ref, b_ref, o_ref):
    nb = x_ref.shape[0] // 192
    x5 = x_ref[...].reshape(nb, 3, 8, 8, 64)
    x5 = x5.astype(jnp.bfloat16).astype(jnp.float32)
    s = jnp.sum(x5 * w_ref[...][None, :, None, :, :], axis=(1, 3))
    r = jnp.dot(s.reshape(nb * 8, 64), s_ref[...],
                preferred_element_type=jnp.float32)
    o_ref[...] = r + b_ref[...]


def kernel(conv_1_w, conv_1_b, conv_2_w, conv_2_b, conv_3_w, conv_3_b, x):
    N, C, H, W = x.shape
    Ho, Wo = H // 8, W // 8

    t = jnp.einsum("uckl,cvij->uvklij", conv_2_w, conv_1_w)
    wfull = jnp.einsum("upq,uvklij->vpkiqlj", conv_3_w[0], t).reshape(C, 8, 8)
    wfull = wfull.astype(jnp.bfloat16).astype(jnp.float32)
    wrow = jnp.tile(wfull, (1, 1, Wo))

    w3s = conv_3_w[0].sum(axis=(1, 2))
    bfull = (jnp.einsum("c,uckl,u->", conv_1_b, conv_2_w, w3s)
             + conv_2_b @ w3s + conv_3_b[0]).reshape(1, 1).astype(jnp.float32)

    S = (jnp.arange(W)[:, None] // 8 ==
         jnp.arange(Wo)[None, :]).astype(jnp.float32)

    xf = x.reshape(N * C * H, W)

    NB = 128
    while N % NB:
        NB //= 2
    out = pl.pallas_call(
        _fused_body,
        out_shape=jax.ShapeDtypeStruct((N * Ho, Wo), jnp.float32),
        grid_spec=pltpu.PrefetchScalarGridSpec(
            num_scalar_prefetch=0,
            grid=(N // NB,),
            in_specs=[
                pl.BlockSpec((NB * C * H, W), lambda i: (i, 0)),
                pl.BlockSpec((C, 8, W), lambda i: (0, 0, 0)),
                pl.BlockSpec((W, Wo), lambda i: (0, 0)),
                pl.BlockSpec((1, 1), lambda i: (0, 0)),
            ],
            out_specs=pl.BlockSpec((NB * Ho, Wo), lambda i: (i, 0)),
        ),
        compiler_params=pltpu.CompilerParams(
            dimension_semantics=("arbitrary",),
            vmem_limit_bytes=64 * 1024 * 1024,
        ),
    )(xf, wrow, S, bfull)

    return out.reshape(N, 1, Ho, Wo).astype(x.dtype)


# speedup vs baseline: 1.0682x; 1.0004x over previous
import jax
import jax.numpy as jnp
from jax.experimental import pallas as pl
from jax.experimental.pallas import tpu as pltpu


def _fused_body(x_ref, w_ref, s_ref, b_ref, o_ref):
    nb = x_ref.shape[0] // 192
    x5 = x_ref[...].reshape(nb, 3, 8, 8, 64)
    x5 = x5.astype(jnp.bfloat16).astype(jnp.float32)
    s = jnp.sum(x5 * w_ref[...][None, :, None, :, :], axis=(1, 3))
    r = jnp.dot(s.reshape(nb * 8, 64), s_ref[...],
                preferred_element_type=jnp.float32)
    o_ref[...] = r + b_ref[...]


def kernel(conv_1_w, conv_1_b, conv_2_w, conv_2_b, conv_3_w, conv_3_b, x):
    N, C, H, W = x.shape
    Ho, Wo = H // 8, W // 8

    t = jnp.einsum("uckl,cvij->uvklij", conv_2_w, conv_1_w)
    wfull = jnp.einsum("upq,uvklij->vpkiqlj", conv_3_w[0], t).reshape(C, 8, 8)
    wfull = wfull.astype(jnp.bfloat16).astype(jnp.float32)
    wrow = jnp.tile(wfull, (1, 1, Wo))

    w3s = conv_3_w[0].sum(axis=(1, 2))
    bfull = (jnp.einsum("c,uckl,u->", conv_1_b, conv_2_w, w3s)
             + conv_2_b @ w3s + conv_3_b[0]).reshape(1, 1).astype(jnp.float32)

    S = (jnp.arange(W)[:, None] // 8 ==
         jnp.arange(Wo)[None, :]).astype(jnp.float32)

    xf = x.reshape(N * C * H, W)

    NB = 128
    while N % NB:
        NB //= 2
    out = pl.pallas_call(
        _fused_body,
        out_shape=jax.ShapeDtypeStruct((N * Ho, Wo), jnp.float32),
        grid_spec=pltpu.PrefetchScalarGridSpec(
            num_scalar_prefetch=0,
            grid=(N // NB,),
            in_specs=[
                pl.BlockSpec((NB * C * H, W), lambda i: (i, 0)),
                pl.BlockSpec((C, 8, W), lambda i: (0, 0, 0)),
                pl.BlockSpec((W, Wo), lambda i: (0, 0)),
                pl.BlockSpec((1, 1), lambda i: (0, 0)),
            ],
            out_specs=pl.BlockSpec((NB * Ho, Wo), lambda i: (i, 0)),
        ),
        compiler_params=pltpu.CompilerParams(
            dimension_semantics=("parallel",),
            vmem_limit_bytes=64 * 1024 * 1024,
        ),
    )(xf, wrow, S, bfull)

    return out.reshape(N, 1, Ho, Wo).astype(x.dtype)
